# Initial kernel scaffold; baseline (speedup 1.0000x reference)
import jax
import jax.numpy as jnp
from jax.experimental import pallas as pl


def kernel(heat, wh, reg):
    raise NotImplementedError("write your pallas kernel here")


# speedup vs baseline: 2.4771x; 2.4771x over previous
import jax
import jax.numpy as jnp
from jax import lax
from jax.experimental import pallas as pl

_K = 100


def _nms_kernel(h_ref, o_ref):
    h = h_ref[0]
    neg = jnp.full_like(h[:, :1, :], -jnp.inf)
    vm = jnp.maximum(h, jnp.concatenate([h[:, 1:, :], neg], axis=1))
    vm = jnp.maximum(vm, jnp.concatenate([neg, h[:, :-1, :]], axis=1))
    negc = jnp.full_like(vm[:, :, :1], -jnp.inf)
    hm = jnp.maximum(vm, jnp.concatenate([vm[:, :, 1:], negc], axis=2))
    hm = jnp.maximum(hm, jnp.concatenate([negc, vm[:, :, :-1]], axis=2))
    keep = (hm == h).astype(h.dtype)
    o_ref[0] = h * keep


def _nms(heat):
    b, c, hh, ww = heat.shape
    return pl.pallas_call(
        _nms_kernel,
        grid=(b,),
        in_specs=[pl.BlockSpec((1, c, hh, ww), lambda i: (i, 0, 0, 0))],
        out_specs=pl.BlockSpec((1, c, hh, ww), lambda i: (i, 0, 0, 0)),
        out_shape=jax.ShapeDtypeStruct(heat.shape, heat.dtype),
    )(heat)


def kernel(heat, wh, reg):
    batch, cat, height, width = heat.shape
    hw = height * width
    nmsed = _nms(heat)

    flat = nmsed.reshape(batch, cat * hw)
    scores, flat_inds = lax.top_k(flat, _K)
    clses = (flat_inds // hw).astype(jnp.int32)
    inds = flat_inds % hw
    ys = (inds // width).astype(jnp.float32)
    xs = (inds % width).astype(jnp.float32)

    reg_f = reg.transpose(0, 2, 3, 1).reshape(batch, hw, 2)
    wh_f = wh.transpose(0, 2, 3, 1).reshape(batch, hw, 2)
    reg_g = jnp.take_along_axis(reg_f, inds[:, :, None], axis=1)
    wh_g = jnp.take_along_axis(wh_f, inds[:, :, None], axis=1)

    xs = xs[:, :, None] + reg_g[:, :, 0:1]
    ys = ys[:, :, None] + reg_g[:, :, 1:2]
    bboxes = jnp.concatenate([
        xs - wh_g[..., 0:1] / 2,
        ys - wh_g[..., 1:2] / 2,
        xs + wh_g[..., 0:1] / 2,
        ys + wh_g[..., 1:2] / 2,
    ], axis=2)
    detections = jnp.concatenate(
        [bboxes, scores[:, :, None], clses[:, :, None].astype(jnp.float32)],
        axis=2)
    return detections


# speedup vs baseline: 6.6249x; 2.6744x over previous
import jax
import jax.numpy as jnp
from jax import lax
from jax.experimental import pallas as pl

_K = 100


_CC = 16


def _nms_kernel(h_ref, ov_ref, oi_ref):
    h = h_ref[0]
    x_e = h[:, :, 0, :]
    x_o = h[:, :, 1, :]

    neg = jnp.full_like(x_e[:, :1, :], -jnp.inf)
    x_o_up = jnp.concatenate([neg, x_o[:, :-1, :]], axis=1)
    x_e_dn = jnp.concatenate([x_e[:, 1:, :], neg], axis=1)
    vm_e = jnp.maximum(jnp.maximum(x_e, x_o), x_o_up)
    vm_o = jnp.maximum(jnp.maximum(x_o, x_e), x_e_dn)

    negc = jnp.full_like(x_e[:, :, :1], -jnp.inf)

    def h3(v):
        l = jnp.concatenate([v[:, :, 1:], negc], axis=2)
        r = jnp.concatenate([negc, v[:, :, :-1]], axis=2)
        return jnp.maximum(jnp.maximum(v, l), r)

    hm_e = h3(vm_e)
    hm_o = h3(vm_o)
    nms_e = x_e * (hm_e == x_e).astype(x_e.dtype)
    nms_o = x_o * (hm_o == x_o).astype(x_o.dtype)

    i_iota = lax.broadcasted_iota(jnp.int32, nms_e.shape, 1)
    c_iota = lax.broadcasted_iota(jnp.int32, nms_e.shape, 2)
    fe = (2 * i_iota) * 128 + c_iota
    v1 = jnp.maximum(nms_e, nms_o)
    i1 = jnp.where(nms_e >= nms_o, fe, fe + 128).astype(jnp.float32)

    r_i = lax.broadcasted_iota(jnp.int32, (128, 64), 0)
    c_j = lax.broadcasted_iota(jnp.int32, (128, 64), 1)
    e_sel = (r_i == 2 * c_j).astype(jnp.float32)
    o_sel = (r_i == 2 * c_j + 1).astype(jnp.float32)

    v2 = v1.reshape(_CC * 64, 128)
    i2 = i1.reshape(_CC * 64, 128)
    c = jnp.dot(v2, e_sel, preferred_element_type=jnp.float32)
    d = jnp.dot(v2, o_sel, preferred_element_type=jnp.float32)
    ic = jnp.dot(i2, e_sel, preferred_element_type=jnp.float32)
    io = jnp.dot(i2, o_sel, preferred_element_type=jnp.float32)

    ov_ref[0] = jnp.maximum(c, d)
    oi_ref[0] = jnp.where(c >= d, ic, io).astype(jnp.int32)


def _nms_compact(heat):
    b, cat, hh, ww = heat.shape
    hr = heat.reshape(b, cat, hh // 2, 2, ww)
    nrow = cat * (hh // 2)
    crow = _CC * (hh // 2)
    return pl.pallas_call(
        _nms_kernel,
        grid=(b, cat // _CC),
        in_specs=[pl.BlockSpec((1, _CC, hh // 2, 2, ww),
                               lambda i, j: (i, j, 0, 0, 0))],
        out_specs=[
            pl.BlockSpec((1, crow, ww // 2), lambda i, j: (i, j, 0)),
            pl.BlockSpec((1, crow, ww // 2), lambda i, j: (i, j, 0)),
        ],
        out_shape=[
            jax.ShapeDtypeStruct((b, nrow, ww // 2), heat.dtype),
            jax.ShapeDtypeStruct((b, nrow, ww // 2), jnp.int32),
        ],
    )(hr)


def kernel(heat, wh, reg):
    batch, cat, height, width = heat.shape
    hw = height * width
    vals, idxs = _nms_compact(heat)
    ncand = cat * (height // 2) * (width // 2)

    scores, ci = lax.top_k(vals.reshape(batch, ncand), _K)
    clses = (ci // ((height // 2) * (width // 2))).astype(jnp.int32)
    inds = jnp.take_along_axis(idxs.reshape(batch, ncand), ci, axis=1)
    ys = (inds // width).astype(jnp.float32)
    xs = (inds % width).astype(jnp.float32)

    reg_f = reg.transpose(0, 2, 3, 1).reshape(batch, hw, 2)
    wh_f = wh.transpose(0, 2, 3, 1).reshape(batch, hw, 2)
    reg_g = jnp.take_along_axis(reg_f, inds[:, :, None], axis=1)
    wh_g = jnp.take_along_axis(wh_f, inds[:, :, None], axis=1)

    xs = xs[:, :, None] + reg_g[:, :, 0:1]
    ys = ys[:, :, None] + reg_g[:, :, 1:2]
    bboxes = jnp.concatenate([
        xs - wh_g[..., 0:1] / 2,
        ys - wh_g[..., 1:2] / 2,
        xs + wh_g[..., 0:1] / 2,
        ys + wh_g[..., 1:2] / 2,
    ], axis=2)
    detections = jnp.concatenate(
        [bboxes, scores[:, :, None], clses[:, :, None].astype(jnp.float32)],
        axis=2)
    return detections


# speedup vs baseline: 7.3192x; 1.1048x over previous
import jax
import jax.numpy as jnp
from jax import lax
from jax.experimental import pallas as pl
from jax.experimental.pallas import tpu as pltpu

_K = 100
_CC = 16
_NCAT = 80
_NROW = _NCAT * 64
_CROW = _CC * 64


def _decode_kernel(h_ref, os_ref, oi_ref, oc_ref, cand_ref, cidx_ref):
    j = pl.program_id(1)
    nchunk = pl.num_programs(1)

    h = h_ref[0]
    x_e = h[:, :, 0, :]
    x_o = h[:, :, 1, :]

    neg = jnp.full_like(x_e[:, :1, :], -jnp.inf)
    x_o_up = jnp.concatenate([neg, x_o[:, :-1, :]], axis=1)
    x_e_dn = jnp.concatenate([x_e[:, 1:, :], neg], axis=1)
    vm_e = jnp.maximum(jnp.maximum(x_e, x_o), x_o_up)
    vm_o = jnp.maximum(jnp.maximum(x_o, x_e), x_e_dn)

    negc = jnp.full_like(x_e[:, :, :1], -jnp.inf)

    def h3(v):
        l = jnp.concatenate([v[:, :, 1:], negc], axis=2)
        r = jnp.concatenate([negc, v[:, :, :-1]], axis=2)
        return jnp.maximum(jnp.maximum(v, l), r)

    nms_e = x_e * (h3(vm_e) == x_e).astype(x_e.dtype)
    nms_o = x_o * (h3(vm_o) == x_o).astype(x_o.dtype)

    i_iota = lax.broadcasted_iota(jnp.int32, nms_e.shape, 1)
    c_iota = lax.broadcasted_iota(jnp.int32, nms_e.shape, 2)
    fe = (2 * i_iota) * 128 + c_iota
    v1 = jnp.maximum(nms_e, nms_o)
    i1 = jnp.where(nms_e >= nms_o, fe, fe + 128).astype(jnp.float32)

    r_i = lax.broadcasted_iota(jnp.int32, (128, 64), 0)
    c_j = lax.broadcasted_iota(jnp.int32, (128, 64), 1)
    e_sel = (r_i == 2 * c_j).astype(jnp.float32)
    o_sel = (r_i == 2 * c_j + 1).astype(jnp.float32)

    v2 = v1.reshape(_CROW, 128)
    i2 = i1.reshape(_CROW, 128)
    c = jnp.dot(v2, e_sel, preferred_element_type=jnp.float32)
    d = jnp.dot(v2, o_sel, preferred_element_type=jnp.float32)
    ic = jnp.dot(i2, e_sel, preferred_element_type=jnp.float32)
    io = jnp.dot(i2, o_sel, preferred_element_type=jnp.float32)

    cand_ref[pl.ds(j * _CROW, _CROW), :] = jnp.maximum(c, d)
    cidx_ref[pl.ds(j * _CROW, _CROW), :] = jnp.where(c >= d, ic, io).astype(jnp.int32)

    @pl.when(j == nchunk - 1)
    def _extract():
        rowmax = jnp.max(cand_ref[...].reshape(_NROW // 128, 128, 64), axis=-1)
        fi = (lax.broadcasted_iota(jnp.int32, rowmax.shape, 0) * 128
              + lax.broadcasted_iota(jnp.int32, rowmax.shape, 1))
        kio = lax.broadcasted_iota(jnp.int32, (1, 128), 1)
        lane = lax.broadcasted_iota(jnp.int32, (1, 64), 1)
        big = jnp.int32(1 << 30)

        def body(k, carry):
            rmax, sc, iv, cv = carry
            m = jnp.max(rmax)
            pos = jnp.min(jnp.where(rmax == m, fi, big))
            row = cand_ref[pl.ds(pos, 1), :]
            lpos = jnp.min(jnp.where(row == m, lane, 64))
            mask = (row == m) & (lane == lpos)
            myidx = jnp.sum(jnp.where(mask, cidx_ref[pl.ds(pos, 1), :], 0))
            new_row = jnp.where(mask, 0.0, row)
            cand_ref[pl.ds(pos, 1), :] = new_row
            rmax = jnp.where(fi == pos, jnp.max(new_row), rmax)
            sc = jnp.where(kio == k, m, sc)
            iv = jnp.where(kio == k, myidx, iv)
            cv = jnp.where(kio == k, pos // 64, cv)
            return rmax, sc, iv, cv

        init = (rowmax,
                jnp.zeros((1, 128), jnp.float32),
                jnp.zeros((1, 128), jnp.int32),
                jnp.zeros((1, 128), jnp.int32))
        _, sc, iv, cv = lax.fori_loop(0, _K, body, init)
        os_ref[0] = sc
        oi_ref[0] = iv
        oc_ref[0] = cv


def _decode_topk(heat):
    b, cat, hh, ww = heat.shape
    hr = heat.reshape(b, cat, hh // 2, 2, ww)
    return pl.pallas_call(
        _decode_kernel,
        grid=(b, cat // _CC),
        in_specs=[pl.BlockSpec((1, _CC, hh // 2, 2, ww),
                               lambda i, j: (i, j, 0, 0, 0))],
        out_specs=[
            pl.BlockSpec((1, 1, 128), lambda i, j: (i, 0, 0)),
            pl.BlockSpec((1, 1, 128), lambda i, j: (i, 0, 0)),
            pl.BlockSpec((1, 1, 128), lambda i, j: (i, 0, 0)),
        ],
        out_shape=[
            jax.ShapeDtypeStruct((b, 1, 128), jnp.float32),
            jax.ShapeDtypeStruct((b, 1, 128), jnp.int32),
            jax.ShapeDtypeStruct((b, 1, 128), jnp.int32),
        ],
        scratch_shapes=[
            pltpu.VMEM((_NROW, 64), jnp.float32),
            pltpu.VMEM((_NROW, 64), jnp.int32),
        ],
    )(hr)


def kernel(heat, wh, reg):
    batch, cat, height, width = heat.shape
    hw = height * width
    sc, iv, cv = _decode_topk(heat)
    scores = sc[:, 0, :_K]
    inds = iv[:, 0, :_K]
    clses = cv[:, 0, :_K]
    ys = (inds // width).astype(jnp.float32)
    xs = (inds % width).astype(jnp.float32)

    reg_f = reg.transpose(0, 2, 3, 1).reshape(batch, hw, 2)
    wh_f = wh.transpose(0, 2, 3, 1).reshape(batch, hw, 2)
    reg_g = jnp.take_along_axis(reg_f, inds[:, :, None], axis=1)
    wh_g = jnp.take_along_axis(wh_f, inds[:, :, None], axis=1)

    xs = xs[:, :, None] + reg_g[:, :, 0:1]
    ys = ys[:, :, None] + reg_g[:, :, 1:2]
    bboxes = jnp.concatenate([
        xs - wh_g[..., 0:1] / 2,
        ys - wh_g[..., 1:2] / 2,
        xs + wh_g[..., 0:1] / 2,
        ys + wh_g[..., 1:2] / 2,
    ], axis=2)
    detections = jnp.concatenate(
        [bboxes, scores[:, :, None], clses[:, :, None].astype(jnp.float32)],
        axis=2)
    return detections
